# baseline (speedup 1.0000x reference)
import math

import jax
import jax.numpy as jnp
import numpy as np
from jax.experimental import pallas as pl
from jax.experimental.pallas import tpu as pltpu

D_POSITION = 1024
D_CONTENT = 4096
NUM_TILES = 64
SEQ = 512
ROWS = 128


def _sinusoidal_pe(seq_len, d_model):
    pe = np.zeros((seq_len, d_model), dtype=np.float32)
    position = np.arange(0, seq_len, dtype=np.float32)[:, None]
    div_term = np.exp(
        np.arange(0, d_model, 2, dtype=np.float32) * (-math.log(10000.0) / d_model)
    )
    pe[:, 0::2] = np.sin(position * div_term)
    pe[:, 1::2] = np.cos(position * div_term)
    return pe


def _router_body(wts_ref, pe_ref, x_ref, sigp_ref, sigc_ref,
                 scores_ref, idx_ref, posb_ref):
    b = pl.program_id(0)
    j = pl.program_id(1)
    pw = wts_ref[0]
    cw = wts_ref[1]

    @pl.when(b == 0)
    def _():
        posb_ref[j] = jax.lax.dot_general(
            pe_ref[0], sigp_ref[...],
            (((1,), (0,)), ((), ())),
            preferred_element_type=jnp.float32,
        )

    content = jax.lax.dot_general(
        x_ref[0, 0], sigc_ref[...],
        (((1,), (0,)), ((), ())),
        preferred_element_type=jnp.float32,
    )
    scores = cw * content + pw * posb_ref[j]
    scores_ref[0, 0] = scores

    mx = jnp.max(scores, axis=-1, keepdims=True)
    iota = jax.lax.broadcasted_iota(jnp.int32, scores.shape, 1)
    cand = jnp.where(scores == mx, iota, NUM_TILES)
    idx = jnp.min(cand, axis=-1)
    idx_ref[0, 0] = idx.reshape(1, ROWS)


def kernel(x, positions, signatures, position_weight, content_weight):
    del positions
    batch, seq, _ = x.shape
    pe = jnp.asarray(_sinusoidal_pe(seq, D_POSITION)).reshape(
        seq // ROWS, ROWS, D_POSITION)
    sig_pos = signatures[:, :D_POSITION].T
    sig_con = signatures[:, D_POSITION:].T

    pw = jax.nn.sigmoid(position_weight)
    cw = jax.nn.sigmoid(content_weight)
    total = pw + cw
    wts = jnp.stack([pw / total, cw / total])

    nj = seq // ROWS
    x4 = x.reshape(batch, nj, ROWS, D_CONTENT)

    scores4, idx4 = pl.pallas_call(
        _router_body,
        grid=(batch, nj),
        in_specs=[
            pl.BlockSpec(memory_space=pltpu.SMEM),
            pl.BlockSpec((1, ROWS, D_POSITION), lambda b, j: (j, 0, 0)),
            pl.BlockSpec((1, 1, ROWS, D_CONTENT), lambda b, j: (b, j, 0, 0)),
            pl.BlockSpec((D_POSITION, NUM_TILES), lambda b, j: (0, 0)),
            pl.BlockSpec((D_CONTENT, NUM_TILES), lambda b, j: (0, 0)),
        ],
        out_specs=[
            pl.BlockSpec((1, 1, ROWS, NUM_TILES), lambda b, j: (b, j, 0, 0)),
            pl.BlockSpec((1, 1, 1, ROWS), lambda b, j: (b, j, 0, 0)),
        ],
        out_shape=[
            jax.ShapeDtypeStruct((batch, nj, ROWS, NUM_TILES), jnp.float32),
            jax.ShapeDtypeStruct((batch, nj, 1, ROWS), jnp.int32),
        ],
        scratch_shapes=[pltpu.VMEM((nj, ROWS, NUM_TILES), jnp.float32)],
    )(wts, pe, x4, sig_pos, sig_con)

    scores = scores4.reshape(batch, seq, NUM_TILES)
    indices = idx4.reshape(batch, seq)
    return indices, scores


# speedup vs baseline: 1.2574x; 1.2574x over previous
import math

import jax
import jax.numpy as jnp
import numpy as np
from jax.experimental import pallas as pl
from jax.experimental.pallas import tpu as pltpu

D_POSITION = 1024
D_CONTENT = 4096
NUM_TILES = 64
SEQ = 512
ROWS = 512


def _sinusoidal_pe(seq_len, d_model):
    pe = np.zeros((seq_len, d_model), dtype=np.float32)
    position = np.arange(0, seq_len, dtype=np.float32)[:, None]
    div_term = np.exp(
        np.arange(0, d_model, 2, dtype=np.float32) * (-math.log(10000.0) / d_model)
    )
    pe[:, 0::2] = np.sin(position * div_term)
    pe[:, 1::2] = np.cos(position * div_term)
    return pe


def _router_body(wts_ref, pe_ref, x_ref, sigp_ref, sigc_ref,
                 scores_ref, idx_ref, posb_ref):
    b = pl.program_id(0)
    j = pl.program_id(1)
    pw = wts_ref[0]
    cw = wts_ref[1]

    @pl.when(b == 0)
    def _():
        posb_ref[j] = jax.lax.dot_general(
            pe_ref[0], sigp_ref[...],
            (((1,), (0,)), ((), ())),
            preferred_element_type=jnp.float32,
        )

    content = jax.lax.dot_general(
        x_ref[0, 0], sigc_ref[...],
        (((1,), (0,)), ((), ())),
        preferred_element_type=jnp.float32,
    )
    scores = cw * content + pw * posb_ref[j]
    scores_ref[0, 0] = scores

    mx = jnp.max(scores, axis=-1, keepdims=True)
    iota = jax.lax.broadcasted_iota(jnp.int32, scores.shape, 1)
    cand = jnp.where(scores == mx, iota, NUM_TILES)
    idx = jnp.min(cand, axis=-1)
    idx_ref[0, 0] = idx.reshape(1, ROWS)


def kernel(x, positions, signatures, position_weight, content_weight):
    del positions
    batch, seq, _ = x.shape
    pe = jnp.asarray(_sinusoidal_pe(seq, D_POSITION)).reshape(
        seq // ROWS, ROWS, D_POSITION)
    sig_pos = signatures[:, :D_POSITION].T
    sig_con = signatures[:, D_POSITION:].T

    pw = jax.nn.sigmoid(position_weight)
    cw = jax.nn.sigmoid(content_weight)
    total = pw + cw
    wts = jnp.stack([pw / total, cw / total])

    nj = seq // ROWS
    x4 = x.reshape(batch, nj, ROWS, D_CONTENT)

    scores4, idx4 = pl.pallas_call(
        _router_body,
        grid=(batch, nj),
        in_specs=[
            pl.BlockSpec(memory_space=pltpu.SMEM),
            pl.BlockSpec((1, ROWS, D_POSITION), lambda b, j: (j, 0, 0)),
            pl.BlockSpec((1, 1, ROWS, D_CONTENT), lambda b, j: (b, j, 0, 0)),
            pl.BlockSpec((D_POSITION, NUM_TILES), lambda b, j: (0, 0)),
            pl.BlockSpec((D_CONTENT, NUM_TILES), lambda b, j: (0, 0)),
        ],
        out_specs=[
            pl.BlockSpec((1, 1, ROWS, NUM_TILES), lambda b, j: (b, j, 0, 0)),
            pl.BlockSpec((1, 1, 1, ROWS), lambda b, j: (b, j, 0, 0)),
        ],
        out_shape=[
            jax.ShapeDtypeStruct((batch, nj, ROWS, NUM_TILES), jnp.float32),
            jax.ShapeDtypeStruct((batch, nj, 1, ROWS), jnp.int32),
        ],
        scratch_shapes=[pltpu.VMEM((nj, ROWS, NUM_TILES), jnp.float32)],
    )(wts, pe, x4, sig_pos, sig_con)

    scores = scores4.reshape(batch, seq, NUM_TILES)
    indices = idx4.reshape(batch, seq)
    return indices, scores
